# baseline (speedup 1.0000x reference)
import functools

import jax
import jax.numpy as jnp
import numpy as np
from jax import lax
from jax.experimental import pallas as pl
from jax.experimental.pallas import tpu as pltpu
from jax.experimental.pallas import tpu_sc as plsc

G = 1024
U = 20
L = 50
D = 64
FACTOR = 0.5

NC = 2
NS = 16
NW = NC * NS
GPW = G // NW

CHUNK = 100
CPAD = 104
UPC = CHUNK // L
NCHUNK = (U * L) // CHUNK
TCH = GPW * NCHUNK
NBUF = 10
NK = D // 16
UPAD = 24


def _out_pos(d):
    half, e = divmod(d, 32)
    return 32 * half + (16 if e % 2 else 0) + e // 2


_OUT_POS = np.array([_out_pos(d) for d in range(D)], dtype=np.int32)


def _body(gbi_hbm, cnt_hbm, gu_hbm, tgt_hbm, sim_hbm, utab_hbm, itab_hbm,
          out_hbm,
          idx_v, cnt_v, rows_v, gu_v, tgt_v, trows_v, urows_v, srows_v,
          out_v, sem0, sem1, sem2, sem3, sem4, sem5, sem6, sem7, sem8,
          sem9, semg, sems_t):
    sems = [sem0, sem1, sem2, sem3, sem4, sem5, sem6, sem7, sem8, sem9]
    wid = lax.axis_index("s") * NC + lax.axis_index("c")
    gbase = wid * GPW

    pltpu.sync_copy(gu_hbm.at[pl.ds(gbase * UPAD, GPW * UPAD)], gu_v)
    pltpu.sync_copy(tgt_hbm.at[pl.ds(gbase, GPW)], tgt_v)
    cp_t = pltpu.async_copy(sim_hbm.at[tgt_v], trows_v, semg)
    pltpu.sync_copy(gbi_hbm.at[pl.ds(wid * TCH * CPAD, TCH * CPAD)], idx_v)
    pltpu.sync_copy(cnt_hbm.at[pl.ds(gbase * U * L, GPW * U * L)], cnt_v)
    cp_t.wait()

    def urows_start(gl, pb):
        pltpu.async_copy(utab_hbm.at[gu_v.at[pl.ds(gl * UPAD, UPAD)]],
                         urows_v.at[pb], sems_t)
        pltpu.async_copy(sim_hbm.at[gu_v.at[pl.ds(gl * UPAD, UPAD)]],
                         srows_v.at[pb], sems_t)

    def urows_wait(gl, pb):
        pltpu.make_async_copy(utab_hbm.at[gu_v.at[pl.ds(gl * UPAD, UPAD)]],
                              urows_v.at[pb], sems_t).wait()
        pltpu.make_async_copy(sim_hbm.at[gu_v.at[pl.ds(gl * UPAD, UPAD)]],
                              srows_v.at[pb], sems_t).wait()

    def chunk_start(c, bi):
        return pltpu.async_copy(itab_hbm.at[idx_v.at[pl.ds(c * CPAD, CPAD)]],
                                rows_v.at[bi], sems[bi])

    def chunk_wait(c, bi):
        pltpu.make_async_copy(itab_hbm.at[idx_v.at[pl.ds(c * CPAD, CPAD)]],
                              rows_v.at[bi], sems[bi]).wait()

    urows_start(0, 0)
    for bi in range(NBUF):
        chunk_start(bi, bi)

    zeros4 = tuple(jnp.zeros((16,), jnp.float32) for _ in range(NK))

    @pl.loop(0, TCH // NBUF, init_carry=zeros4)
    def _super(si, og):
        og = list(og)
        for bi in range(NBUF):
            c = si * NBUF + bi
            gl = lax.div(c, NCHUNK)
            j = lax.rem(c, NCHUNK)
            pb = lax.rem(gl, 2)

            @pl.when(j == 0)
            def _():
                urows_wait(gl, pb)

                @pl.when(gl + 1 < GPW)
                def _():
                    urows_start(gl + 1, 1 - pb)

            chunk_wait(c, bi)

            ts = [trows_v[gl, pl.ds(16 * k, 16)] for k in range(NK)]
            for u2 in range(UPC):
                cbase = gl * (U * L) + (UPC * lax.rem(c, NCHUNK) + u2) * L
                u = UPC * j + u2
                rbase = u2 * L

                def _row_acc(acc, r, cc):
                    a0, b0 = plsc.unpack(
                        rows_v[bi, r, pl.ds(0, 32)],
                        format=plsc.PackFormat.INTERLEAVED)
                    a1, b1 = plsc.unpack(
                        rows_v[bi, r, pl.ds(32, 32)],
                        format=plsc.PackFormat.INTERLEAVED)
                    return [acc[0] + a0 * cc, acc[1] + b0 * cc,
                            acc[2] + a1 * cc, acc[3] + b1 * cc]

                def _tblock(t, acc):
                    cv = cnt_v[pl.ds(cbase + t * 16, 16)]
                    acc = list(acc)
                    for i in range(16):
                        acc = _row_acc(acc, rbase + t * 16 + i, cv[i])
                    return tuple(acc)

                acc = list(lax.fori_loop(0, 3, _tblock, zeros4))
                cvt = cnt_v[pl.ds(cbase + 34, 16)]
                for i in range(2):
                    acc = _row_acc(acc, rbase + 48 + i, cvt[14 + i])

                sv = ts[0] * srows_v[pb, u, pl.ds(0, 16)]
                for k in range(1, NK):
                    sv = sv + ts[k] * srows_v[pb, u, pl.ds(16 * k, 16)]
                s = jnp.sum(sv) * FACTOR
                ue0, ue1 = plsc.unpack(urows_v[pb, u, pl.ds(0, 32)],
                                       format=plsc.PackFormat.INTERLEAVED)
                ue2, ue3 = plsc.unpack(urows_v[pb, u, pl.ds(32, 32)],
                                       format=plsc.PackFormat.INTERLEAVED)
                for k, ue in enumerate((ue0, ue1, ue2, ue3)):
                    og[k] = og[k] + acc[k] * ue * s

            @pl.when(j == NCHUNK - 1)
            def _():
                for k in range(NK):
                    out_v[pl.ds(gl * D + 16 * k, 16)] = og[k]

            og = [jnp.where(j == NCHUNK - 1, jnp.zeros((16,), jnp.float32),
                            og[k]) for k in range(NK)]

            @pl.when(c + NBUF < TCH)
            def _():
                chunk_start(c + NBUF, bi)
        return tuple(og)

    pltpu.sync_copy(out_v, out_hbm.at[pl.ds(gbase * D, GPW * D)])


_sc_call = pl.kernel(
    _body,
    out_type=jax.ShapeDtypeStruct((G * D,), jnp.float32),
    mesh=plsc.VectorSubcoreMesh(core_axis_name="c", subcore_axis_name="s",
                                num_cores=NC, num_subcores=NS),
    compiler_params=pltpu.CompilerParams(needs_layout_passes=False,
                                         use_tc_tiling_on_sc=False),
    scratch_types=[
        pltpu.VMEM((TCH * CPAD,), jnp.int32),
        pltpu.VMEM((GPW * U * L,), jnp.float32),
        pltpu.VMEM((NBUF, CPAD, D), jnp.bfloat16),
        pltpu.VMEM((GPW * UPAD,), jnp.int32),
        pltpu.VMEM((GPW,), jnp.int32),
        pltpu.VMEM((GPW, D), jnp.float32),
        pltpu.VMEM((2, UPAD, D), jnp.bfloat16),
        pltpu.VMEM((2, UPAD, D), jnp.float32),
        pltpu.VMEM((GPW * D,), jnp.float32),
        pltpu.SemaphoreType.DMA,
        pltpu.SemaphoreType.DMA,
        pltpu.SemaphoreType.DMA,
        pltpu.SemaphoreType.DMA,
        pltpu.SemaphoreType.DMA,
        pltpu.SemaphoreType.DMA,
        pltpu.SemaphoreType.DMA,
        pltpu.SemaphoreType.DMA,
        pltpu.SemaphoreType.DMA,
        pltpu.SemaphoreType.DMA,
        pltpu.SemaphoreType.DMA,
        pltpu.SemaphoreType.DMA,
    ],
)


@jax.jit
def kernel(group_user, group_behavior_ids, group_behavior_counts,
           target_user, similarity_vec, user_table, item_table):
    gbi = group_behavior_ids.astype(jnp.int32).reshape(G, NCHUNK, CHUNK)
    gbi = jnp.pad(gbi, ((0, 0), (0, 0), (0, CPAD - CHUNK))).reshape(-1)
    cnt = group_behavior_counts.reshape(-1)
    gu = group_user.astype(jnp.int32)
    gu = jnp.pad(gu, ((0, 0), (0, UPAD - U))).reshape(-1)
    tgt = target_user.astype(jnp.int32)
    itab_bf = item_table.astype(jnp.bfloat16)
    utab_bf = user_table.astype(jnp.bfloat16)
    out = _sc_call(gbi, cnt, gu, tgt, similarity_vec, utab_bf, itab_bf)
    return out.reshape(G, D)[:, _OUT_POS]


# speedup vs baseline: 1.8528x; 1.8528x over previous
import functools

import jax
import jax.numpy as jnp
from jax import lax
from jax.experimental import pallas as pl
from jax.experimental.pallas import tpu as pltpu
from jax.experimental.pallas import tpu_sc as plsc

G = 1024
U = 20
L = 50
D = 64
FACTOR = 0.5

NC = 2
NS = 16
NW = NC * NS
GPW = G // NW

CHUNK = 100
NCHUNK = (U * L) // CHUNK
TCH = GPW * NCHUNK
NBUF = 5
NK = D // 16


def _body(gbi_hbm, cnt_hbm, gu_hbm, tgt_hbm, sim_hbm, utab_hbm, itab_hbm,
          out_hbm,
          idx_v, cnt_v, rows_v, gu_v, tgt_v, trows_v, urows_v, srows_v,
          out_v, sem0, sem1, sem2, sem3, sem4, semg, sems_t):
    sems = [sem0, sem1, sem2, sem3, sem4]
    wid = lax.axis_index("s") * NC + lax.axis_index("c")
    gbase = wid * GPW

    pltpu.sync_copy(gu_hbm.at[pl.ds(gbase, GPW)], gu_v)
    pltpu.sync_copy(tgt_hbm.at[pl.ds(gbase, GPW)], tgt_v)
    cp_t = pltpu.async_copy(sim_hbm.at[tgt_v], trows_v, semg)
    pltpu.sync_copy(gbi_hbm.at[pl.ds(gbase, GPW)], idx_v)
    pltpu.sync_copy(cnt_hbm.at[pl.ds(gbase, GPW)], cnt_v)
    cp_t.wait()

    def urows_start(gl, pb):
        pltpu.async_copy(utab_hbm.at[gu_v.at[gl]], urows_v.at[pb], sems_t)
        pltpu.async_copy(sim_hbm.at[gu_v.at[gl]], srows_v.at[pb], sems_t)

    def urows_wait(gl, pb):
        pltpu.make_async_copy(utab_hbm.at[gu_v.at[gl]], urows_v.at[pb],
                              sems_t).wait()
        pltpu.make_async_copy(sim_hbm.at[gu_v.at[gl]], srows_v.at[pb],
                              sems_t).wait()

    def chunk_start(gl, j, bi):
        return pltpu.async_copy(itab_hbm.at[idx_v.at[gl, j]],
                                rows_v.at[bi], sems[bi])

    urows_start(0, 0)
    for bi in range(NBUF):
        chunk_start(0, bi, bi)

    zeros4 = tuple(jnp.zeros((16,), jnp.float32) for _ in range(NK))

    @pl.loop(0, TCH // NBUF, init_carry=zeros4)
    def _super(si, og):
        og = list(og)
        for bi in range(NBUF):
            c = si * NBUF + bi
            gl = lax.div(c, NCHUNK)
            j = lax.rem(c, NCHUNK)
            pb = lax.rem(gl, 2)

            @pl.when(j == 0)
            def _():
                urows_wait(gl, pb)

                @pl.when(gl + 1 < GPW)
                def _():
                    urows_start(gl + 1, 1 - pb)

            pltpu.make_async_copy(itab_hbm.at[idx_v.at[gl, j]],
                                  rows_v.at[bi], sems[bi]).wait()

            ts = [trows_v[gl, pl.ds(16 * k, 16)] for k in range(NK)]
            for u2 in range(2):
                u = 2 * j + u2
                cbase = u * L
                rbase = u2 * L

                def _tblock(t, acc):
                    cv = cnt_v[gl, pl.ds(cbase + t * 16, 16)]
                    acc = list(acc)
                    for i in range(16):
                        cc = cv[i]
                        for k in range(NK):
                            acc[k] = acc[k] + rows_v[
                                bi, rbase + t * 16 + i,
                                pl.ds(16 * k, 16)] * cc
                    return tuple(acc)

                acc = list(lax.fori_loop(0, 3, _tblock, zeros4))
                cvt = cnt_v[gl, pl.ds(cbase + 34, 16)]
                for i in range(2):
                    cc = cvt[14 + i]
                    for k in range(NK):
                        acc[k] = acc[k] + rows_v[
                            bi, rbase + 48 + i, pl.ds(16 * k, 16)] * cc

                sv = ts[0] * srows_v[pb, u, pl.ds(0, 16)]
                for k in range(1, NK):
                    sv = sv + ts[k] * srows_v[pb, u, pl.ds(16 * k, 16)]
                s = jnp.sum(sv) * FACTOR
                for k in range(NK):
                    og[k] = og[k] + acc[k] * urows_v[
                        pb, u, pl.ds(16 * k, 16)] * s

            @pl.when(j == NCHUNK - 1)
            def _():
                for k in range(NK):
                    out_v[pl.ds(gl * D + 16 * k, 16)] = og[k]

            og = [jnp.where(j == NCHUNK - 1, jnp.zeros((16,), jnp.float32),
                            og[k]) for k in range(NK)]

            @pl.when(c + NBUF < TCH)
            def _():
                cn = c + NBUF
                chunk_start(lax.div(cn, NCHUNK), lax.rem(cn, NCHUNK), bi)
        return tuple(og)

    pltpu.sync_copy(out_v, out_hbm.at[pl.ds(gbase * D, GPW * D)])


_sc_call = pl.kernel(
    _body,
    out_type=jax.ShapeDtypeStruct((G * D,), jnp.float32),
    mesh=plsc.VectorSubcoreMesh(core_axis_name="c", subcore_axis_name="s",
                                num_cores=NC, num_subcores=NS),
    compiler_params=pltpu.CompilerParams(needs_layout_passes=False,
                                         use_tc_tiling_on_sc=False),
    scratch_types=[
        pltpu.VMEM((GPW, NCHUNK, CHUNK), jnp.int32),
        pltpu.VMEM((GPW, U * L), jnp.float32),
        pltpu.VMEM((NBUF, CHUNK, D), jnp.float32),
        pltpu.VMEM((GPW, U), jnp.int32),
        pltpu.VMEM((GPW,), jnp.int32),
        pltpu.VMEM((GPW, D), jnp.float32),
        pltpu.VMEM((2, U, D), jnp.float32),
        pltpu.VMEM((2, U, D), jnp.float32),
        pltpu.VMEM((GPW * D,), jnp.float32),
        pltpu.SemaphoreType.DMA,
        pltpu.SemaphoreType.DMA,
        pltpu.SemaphoreType.DMA,
        pltpu.SemaphoreType.DMA,
        pltpu.SemaphoreType.DMA,
        pltpu.SemaphoreType.DMA,
        pltpu.SemaphoreType.DMA,
    ],
)


@jax.jit
def kernel(group_user, group_behavior_ids, group_behavior_counts,
           target_user, similarity_vec, user_table, item_table):
    gbi = group_behavior_ids.astype(jnp.int32).reshape(G, NCHUNK, CHUNK)
    cnt = group_behavior_counts.reshape(G, U * L)
    gu = group_user.astype(jnp.int32)
    tgt = target_user.astype(jnp.int32)
    out = _sc_call(gbi, cnt, gu, tgt, similarity_vec, user_table, item_table)
    return out.reshape(G, D)


# speedup vs baseline: 1.8922x; 1.0213x over previous
import functools

import jax
import jax.numpy as jnp
from jax import lax
from jax.experimental import pallas as pl
from jax.experimental.pallas import tpu as pltpu
from jax.experimental.pallas import tpu_sc as plsc

G = 1024
U = 20
L = 50
D = 64
FACTOR = 0.5

NC = 2
NS = 16
NW = NC * NS
GPW = G // NW

CHUNK = 100
NCHUNK = (U * L) // CHUNK
TCH = GPW * NCHUNK
NBUF = 5
NK = D // 16


def _body(gbi_hbm, cnt_hbm, gu_hbm, tgt_hbm, sim_hbm, utab_hbm, itab_hbm,
          out_hbm,
          idx_v, cnt_v, rows_v, gu_v, tgt_v, trows_v, urows_v, srows_v,
          out_v, sem0, sem1, sem2, sem3, sem4, semg, sems_t):
    sems = [sem0, sem1, sem2, sem3, sem4]
    wid = lax.axis_index("s") * NC + lax.axis_index("c")
    gbase = wid * GPW

    pltpu.sync_copy(gu_hbm.at[pl.ds(gbase, GPW)], gu_v)
    pltpu.sync_copy(tgt_hbm.at[pl.ds(gbase, GPW)], tgt_v)
    cp_t = pltpu.async_copy(sim_hbm.at[tgt_v], trows_v, semg)
    pltpu.sync_copy(gbi_hbm.at[pl.ds(gbase, GPW)], idx_v)
    pltpu.sync_copy(cnt_hbm.at[pl.ds(gbase, GPW)], cnt_v)
    cp_t.wait()

    def urows_start(gl, pb):
        pltpu.async_copy(utab_hbm.at[gu_v.at[gl]], urows_v.at[pb], sems_t)
        pltpu.async_copy(sim_hbm.at[gu_v.at[gl]], srows_v.at[pb], sems_t)

    def urows_wait(gl, pb):
        pltpu.make_async_copy(utab_hbm.at[gu_v.at[gl]], urows_v.at[pb],
                              sems_t).wait()
        pltpu.make_async_copy(sim_hbm.at[gu_v.at[gl]], srows_v.at[pb],
                              sems_t).wait()

    def chunk_start(gl, j, bi):
        return pltpu.async_copy(itab_hbm.at[idx_v.at[gl, j]],
                                rows_v.at[bi], sems[bi])

    urows_start(0, 0)
    for bi in range(NBUF):
        chunk_start(0, bi, bi)

    zeros4 = tuple(jnp.zeros((16,), jnp.float32) for _ in range(NK))

    @pl.loop(0, TCH // NBUF, init_carry=zeros4)
    def _super(si, og):
        og = list(og)
        for bi in range(NBUF):
            c = si * NBUF + bi
            gl = lax.div(c, NCHUNK)
            j = lax.rem(c, NCHUNK)
            pb = lax.rem(gl, 2)

            @pl.when(j == 0)
            def _():
                urows_wait(gl, pb)

                @pl.when(gl + 1 < GPW)
                def _():
                    urows_start(gl + 1, 1 - pb)

            pltpu.make_async_copy(itab_hbm.at[idx_v.at[gl, j]],
                                  rows_v.at[bi], sems[bi]).wait()

            ts = [trows_v[gl, pl.ds(16 * k, 16)] for k in range(NK)]
            for u2 in range(2):
                u = 2 * j + u2
                cbase = u * L
                rbase = u2 * L

                def _tblock(t, acc):
                    cv = cnt_v[gl, pl.ds(cbase + t * 16, 16)]
                    acc = list(acc)
                    for i in range(16):
                        cc = cv[i]
                        for k in range(NK):
                            acc[k] = acc[k] + rows_v[
                                bi, rbase + t * 16 + i,
                                pl.ds(16 * k, 16)] * cc
                    return tuple(acc)

                acc = list(lax.fori_loop(0, 3, _tblock, zeros4))
                cvt = cnt_v[gl, pl.ds(cbase + 34, 16)]
                for i in range(2):
                    cc = cvt[14 + i]
                    for k in range(NK):
                        acc[k] = acc[k] + rows_v[
                            bi, rbase + 48 + i, pl.ds(16 * k, 16)] * cc

                s = jnp.float32(0.0)
                for k in range(NK):
                    s = s + jnp.sum(ts[k] * srows_v[pb, u, pl.ds(16 * k, 16)])
                s = s * FACTOR
                for k in range(NK):
                    og[k] = og[k] + acc[k] * urows_v[
                        pb, u, pl.ds(16 * k, 16)] * s

            @pl.when(j == NCHUNK - 1)
            def _():
                for k in range(NK):
                    out_v[gl, pl.ds(16 * k, 16)] = og[k]

            og = [jnp.where(j == NCHUNK - 1, jnp.zeros((16,), jnp.float32),
                            og[k]) for k in range(NK)]

            @pl.when(c + NBUF < TCH)
            def _():
                cn = c + NBUF
                chunk_start(lax.div(cn, NCHUNK), lax.rem(cn, NCHUNK), bi)
        return tuple(og)

    pltpu.sync_copy(out_v, out_hbm.at[pl.ds(gbase, GPW)])


_sc_call = pl.kernel(
    _body,
    out_type=jax.ShapeDtypeStruct((G, D), jnp.float32),
    mesh=plsc.VectorSubcoreMesh(core_axis_name="c", subcore_axis_name="s",
                                num_cores=NC, num_subcores=NS),
    compiler_params=pltpu.CompilerParams(needs_layout_passes=False,
                                         use_tc_tiling_on_sc=False),
    scratch_types=[
        pltpu.VMEM((GPW, NCHUNK, CHUNK), jnp.int32),
        pltpu.VMEM((GPW, U * L), jnp.float32),
        pltpu.VMEM((NBUF, CHUNK, D), jnp.float32),
        pltpu.VMEM((GPW, U), jnp.int32),
        pltpu.VMEM((GPW,), jnp.int32),
        pltpu.VMEM((GPW, D), jnp.float32),
        pltpu.VMEM((2, U, D), jnp.float32),
        pltpu.VMEM((2, U, D), jnp.float32),
        pltpu.VMEM((GPW, D), jnp.float32),
        pltpu.SemaphoreType.DMA,
        pltpu.SemaphoreType.DMA,
        pltpu.SemaphoreType.DMA,
        pltpu.SemaphoreType.DMA,
        pltpu.SemaphoreType.DMA,
        pltpu.SemaphoreType.DMA,
        pltpu.SemaphoreType.DMA,
    ],
)


@jax.jit
def kernel(group_user, group_behavior_ids, group_behavior_counts,
           target_user, similarity_vec, user_table, item_table):
    gbi = group_behavior_ids.astype(jnp.int32).reshape(G, NCHUNK, CHUNK)
    cnt = group_behavior_counts.reshape(G, U * L)
    gu = group_user.astype(jnp.int32)
    tgt = target_user.astype(jnp.int32)
    return _sc_call(gbi, cnt, gu, tgt, similarity_vec, user_table, item_table)
